# baseline (speedup 1.0000x reference)
import functools

import jax
import jax.numpy as jnp
from jax import lax
from jax.experimental import pallas as pl
from jax.experimental.pallas import tpu as pltpu
from jax.experimental.pallas import tpu_sc as plsc

N = 10000
E = 160000
D = 128
B = 8
R = 92
C = 16

NC = 2
NS = 16
NW = NC * NS
DUMP = N
N_PAD = 10112
RPS = N_PAD // NS
DEG_ROWS = 320
E_PAD = 163840
EPW = E_PAD // NW

ROW_BLK = 1000


def _make_edge_kernel(tw, K, CB, with_deg, table_bf16=False):
    G = K // 16
    dw = tw // B
    NB = EPW // K
    NCH = NB // CB
    CW = CB * K
    mesh = plsc.VectorSubcoreMesh(core_axis_name="c", subcore_axis_name="s")

    out_type = [jax.ShapeDtypeStruct((NC, N_PAD, 128), jnp.float32)]
    scratch = [
        pltpu.VMEM((2, CW), jnp.int32),
        pltpu.VMEM((2, CW), jnp.int32),
        pltpu.VMEM((2, CW), jnp.int32),
        pltpu.VMEM((2, CW), jnp.float32),
        pltpu.VMEM((R * B,), jnp.float32),
        pltpu.VMEM((2, K, tw // 2) if table_bf16 else (2, K, tw),
                   jnp.int32 if table_bf16 else jnp.float32),
        pltpu.VMEM((2, K, 128), jnp.float32),
        pltpu.VMEM((B, K), jnp.float32),
        pltpu.VMEM_SHARED((N_PAD, 128), jnp.float32),
        pltpu.SemaphoreType.DMA((2,)),
        pltpu.SemaphoreType.DMA((2,)),
    ]
    if with_deg:
        out_type.append(jax.ShapeDtypeStruct((NC, DEG_ROWS, 128), jnp.float32))
        scratch.append(pltpu.VMEM((2, K, 128), jnp.float32))
        scratch.append(pltpu.VMEM_SHARED((DEG_ROWS, 128), jnp.float32))

    @functools.partial(
        pl.kernel,
        out_type=out_type,
        mesh=mesh,
        compiler_params=pltpu.CompilerParams(needs_layout_passes=False),
        scratch_types=scratch,
    )
    def edge_kernel(y_hbm, src_hbm, dst_hbm, et_hbm, p_hbm, comp_hbm,
                    zeros_hbm, *refs):
        if with_deg:
            (out_hbm, deg_hbm, src_v, dst_v, et_v, p_v, comp_v, gbuf, mbuf,
             cbuf, acc_sh, gsem, ssem, dbuf, deg_sh) = refs
        else:
            (out_hbm, src_v, dst_v, et_v, p_v, comp_v, gbuf, mbuf,
             cbuf, acc_sh, gsem, ssem) = refs
        cid = lax.axis_index("c")
        sid = lax.axis_index("s")
        wid = cid * NS + sid
        ebase = wid * EPW

        def stage_chunk(ch, pslot):
            off = pl.multiple_of(ebase + ch * CW, 8)
            pltpu.sync_copy(src_hbm.at[pl.ds(off, CW)], src_v.at[pslot])
            pltpu.sync_copy(dst_hbm.at[pl.ds(off, CW)], dst_v.at[pslot])
            pltpu.sync_copy(et_hbm.at[pl.ds(off, CW)], et_v.at[pslot])
            pltpu.sync_copy(p_hbm.at[pl.ds(off, CW)], p_v.at[pslot])

        def eslice(j, g):
            ch = j // CB
            pslot = lax.rem(ch, 2)
            off = pl.multiple_of((j - ch * CB) * K + g * 16, 8)
            return pslot, off

        pltpu.sync_copy(comp_hbm, comp_v)
        pltpu.sync_copy(zeros_hbm.at[pl.ds(sid * RPS, RPS)],
                        acc_sh.at[pl.ds(sid * RPS, RPS)])
        if with_deg:
            @pl.when(sid == 0)
            def _():
                pltpu.sync_copy(zeros_hbm.at[pl.ds(0, DEG_ROWS)], deg_sh)

        zer = jnp.zeros((16,), jnp.float32)

        def zero_body(e, _):
            for s in range(2):
                for k in range(8):
                    mbuf[s, e, pl.ds(k * 16, 16)] = zer
                    if with_deg:
                        dbuf[s, e, pl.ds(k * 16, 16)] = zer
            return 0
        lax.fori_loop(0, K, zero_body, 0)

        def start_gather(j, slot):
            for g in range(G):
                pslot, off = eslice(j, g)
                ivec = src_v[pslot, pl.ds(off, 16)]
                pltpu.async_copy(y_hbm.at[ivec],
                                 gbuf.at[slot, pl.ds(g * 16, 16)],
                                 gsem.at[slot])

        def wait_gather(slot):
            for g in range(G):
                dummy = src_v[0, pl.ds(g * 16, 16)]
                pltpu.make_async_copy(y_hbm.at[dummy],
                                      gbuf.at[slot, pl.ds(g * 16, 16)],
                                      gsem.at[slot]).wait()

        def compute_batch(j, slot):
            cvecs = None
            for g in range(G):
                pslot, off = eslice(j, g)
                et_g = et_v[pslot, pl.ds(off, 16)].astype(jnp.float32)
                p_g = p_v[pslot, pl.ds(off, 16)]
                pf = et_g * p_g + 91.0 * (1.0 - p_g)
                pf = jnp.minimum(jnp.maximum(pf, 0.0), 91.0) + 0.5
                rel = pf.astype(jnp.int32) * B
                if K == 16:
                    cvecs = [plsc.load_gather(comp_v, [rel + b])
                             for b in range(B)]
                else:
                    for b in range(B):
                        cbuf[b, pl.ds(g * 16, 16)] = plsc.load_gather(
                            comp_v, [rel + b])

            @plsc.parallel_loop(0, K, step=1, unroll=2, carry=jnp.int32(0))
            def edge_body(e, _):
                if K == 16:
                    lane = e
                else:
                    cbase = pl.multiple_of(e & ~15, 8)
                    lane = e & 15
                lanes = jnp.full((16, 1), lane, jnp.int32)
                dnums = lax.GatherDimensionNumbers(
                    offset_dims=(), collapsed_slice_dims=(0,),
                    start_index_map=(0,))

                def bc(b):
                    src = (cvecs[b] if K == 16
                           else cbuf[b, pl.ds(cbase, 16)])
                    return lax.gather(
                        src, lanes, dnums, (1,),
                        mode=lax.GatherScatterMode.PROMISE_IN_BOUNDS)

                cbs = [bc(b) for b in range(B)]
                if table_bf16:
                    himask = jnp.full((16,), -65536, jnp.int32)
                    m = [None] * (dw // 16)
                    for b in range(B):
                        for t in range(dw // 32):
                            v = gbuf[slot, e, pl.ds(b * dw // 2 + t * 16, 16)]
                            av = plsc.bitcast(
                                lax.shift_left(v, jnp.full((16,), 16,
                                                           jnp.int32)),
                                jnp.float32)
                            bv = plsc.bitcast(v & himask, jnp.float32)
                            if b == 0:
                                m[2 * t] = cbs[0] * av
                                m[2 * t + 1] = cbs[0] * bv
                            else:
                                m[2 * t] = m[2 * t] + cbs[b] * av
                                m[2 * t + 1] = m[2 * t + 1] + cbs[b] * bv
                    for k in range(dw // 16):
                        mbuf[slot, e, pl.ds(k * 16, 16)] = m[k]
                else:
                    for k in range(dw // 16):
                        m = cbs[0] * gbuf[slot, e, pl.ds(k * 16, 16)]
                        for b in range(1, B):
                            m = m + cbs[b] * gbuf[slot, e,
                                                  pl.ds(b * dw + k * 16, 16)]
                        mbuf[slot, e, pl.ds(k * 16, 16)] = m
                if with_deg:
                    dbuf[slot, e, pl.ds(0, 16)] = zer
                    dbuf[slot, e, pl.ds(16, 16)] = zer
                return _

        ones_v = jnp.full((16,), 1.0, jnp.float32)
        lane_iota = lax.iota(jnp.int32, 16)

        def scatter_batch(j, slot):
            slotv = jnp.full((16,), slot, jnp.int32)
            for g in range(G):
                pslot, off = eslice(j, g)
                dvec = dst_v[pslot, pl.ds(off, 16)]
                if with_deg:
                    evec = lane_iota + (g * 16)
                    colv = dvec & 31
                    plsc.store_scatter(dbuf, [slotv, evec, colv], ones_v)
                pltpu.async_copy(mbuf.at[slot, pl.ds(g * 16, 16)],
                                 acc_sh.at[dvec], ssem.at[slot], add=True)
                if with_deg:
                    drow = lax.shift_right_logical(dvec, 5)
                    pltpu.async_copy(dbuf.at[slot, pl.ds(g * 16, 16)],
                                     deg_sh.at[drow], ssem.at[slot], add=True)

        def wait_scatter(slot):
            n_waits = G * (2 if with_deg else 1)
            for _w in range(n_waits):
                pltpu.make_async_copy(mbuf.at[slot, pl.ds(0, 16)],
                                      acc_sh.at[lane_iota],
                                      ssem.at[slot]).wait()

        plsc.subcore_barrier()

        stage_chunk(0, 0)
        start_gather(0, 0)

        def pair_body(i, _):
            j = i * 2

            @pl.when(lax.rem(j, CB) == 0)
            def _():
                ch = j // CB

                @pl.when(ch + 1 < NCH)
                def _():
                    stage_chunk(ch + 1, lax.rem(ch + 1, 2))

            wait_gather(0)
            start_gather(j + 1, 1)

            @pl.when(i > 0)
            def _():
                wait_scatter(0)

            compute_batch(j, 0)
            scatter_batch(j, 0)
            wait_gather(1)

            @pl.when(j + 2 < NB)
            def _():
                start_gather(j + 2, 0)

            @pl.when(i > 0)
            def _():
                wait_scatter(1)

            compute_batch(j + 1, 1)
            scatter_batch(j + 1, 1)
            return 0

        lax.fori_loop(0, NB // 2, pair_body, 0)

        wait_scatter(0)
        wait_scatter(1)
        plsc.subcore_barrier()
        pltpu.sync_copy(acc_sh.at[pl.ds(sid * RPS, RPS)],
                        out_hbm.at[cid, pl.ds(sid * RPS, RPS)])
        if with_deg:
            @pl.when(sid == 0)
            def _():
                pltpu.sync_copy(deg_sh, deg_hbm.at[cid])

    return edge_kernel


_edge_l1 = _make_edge_kernel(tw=B * D, K=16, CB=40, with_deg=True,
                             table_bf16=True)
_edge_l2 = _make_edge_kernel(tw=B * C, K=64, CB=16, with_deg=False)


def _dot(a, b):
    return lax.dot_general(a, b, (((1,), (0,)), ((), ())),
                           precision=lax.Precision.HIGHEST,
                           preferred_element_type=jnp.float32)


def _mm1_body(x_ref, w_ref, b_ref, r_ref, y_ref):
    m = _dot(x_ref[...], w_ref[...])
    r_ref[...] = m[:, :D] + b_ref[...]
    y_ref[...] = m[:, D:].astype(jnp.bfloat16)


def _inv_deg(deg):
    return jnp.where(deg > 0, 1.0 / jnp.maximum(deg, 1.0), 0.0)


def _mm2_body(r1_ref, a0_ref, a1_ref, d0_ref, d1_ref, w_ref, b_ref,
              r2_ref, y2_ref):
    inv = _inv_deg(d0_ref[...][:, 0] + d1_ref[...][:, 0])
    a = a0_ref[...] + a1_ref[...]
    h = jnp.maximum(r1_ref[...] + inv[:, None] * a, 0.0)
    m = _dot(h, w_ref[...])
    r2_ref[...] = m[:, :C] + b_ref[...]
    y2_ref[...] = m[:, C:]


def _final_body(r2_ref, c0_ref, c1_ref, d0_ref, d1_ref, o_ref):
    inv = _inv_deg(d0_ref[...][:, 0] + d1_ref[...][:, 0])
    logits = r2_ref[...] + inv[:, None] * (c0_ref[...] + c1_ref[...])
    mx = jnp.max(logits, axis=1, keepdims=True)
    s = logits - mx
    o_ref[...] = s - jnp.log(jnp.sum(jnp.exp(s), axis=1, keepdims=True))


def _row_spec(w):
    return pl.BlockSpec((ROW_BLK, w), lambda i: (i, 0))


def _full_spec(h, w):
    return pl.BlockSpec((h, w), lambda i: (0, 0))


_GRID = N // ROW_BLK

_mm1 = pl.pallas_call(
    _mm1_body,
    grid=(_GRID,),
    in_specs=[_row_spec(D), _full_spec(D, D + B * D), _full_spec(1, D)],
    out_specs=[_row_spec(D), _row_spec(B * D)],
    out_shape=[jax.ShapeDtypeStruct((N, D), jnp.float32),
               jax.ShapeDtypeStruct((N, B * D), jnp.bfloat16)],
)

_mm2 = pl.pallas_call(
    _mm2_body,
    grid=(_GRID,),
    in_specs=[_row_spec(D), _row_spec(D), _row_spec(D),
              _row_spec(16), _row_spec(16),
              _full_spec(D, C + B * C), _full_spec(1, C)],
    out_specs=[_row_spec(C), _row_spec(B * C)],
    out_shape=[jax.ShapeDtypeStruct((N, C), jnp.float32),
               jax.ShapeDtypeStruct((N, B * C), jnp.float32)],
)

_final = pl.pallas_call(
    _final_body,
    grid=(_GRID,),
    in_specs=[_row_spec(C), _row_spec(C), _row_spec(C),
              _row_spec(16), _row_spec(16)],
    out_specs=_row_spec(C),
    out_shape=jax.ShapeDtypeStruct((N, C), jnp.float32),
)


def kernel(sub_edge_index, sub_edge_type, P_vec, entity_emb, bases1, comp1,
           root1, bias1, bases2, comp2, root2, bias2):
    x = entity_emb
    perm = [c0 + x for c0 in range(0, B * D, 32)
            for i in range(16) for x in (i, 16 + i)]
    w1y = jnp.transpose(bases1, (1, 0, 2)).reshape(D, B * D)[:, jnp.array(perm)]
    w1 = jnp.concatenate([root1, w1y], axis=1)
    w2 = jnp.concatenate(
        [root2, jnp.transpose(bases2, (1, 0, 2)).reshape(D, B * C)], axis=1)

    pad = E_PAD - E
    src_p = jnp.concatenate(
        [sub_edge_index[0].astype(jnp.int32), jnp.zeros((pad,), jnp.int32)])
    dst_p = jnp.concatenate(
        [sub_edge_index[1].astype(jnp.int32), jnp.full((pad,), DUMP, jnp.int32)])
    et_p = jnp.concatenate(
        [sub_edge_type.astype(jnp.int32), jnp.zeros((pad,), jnp.int32)])
    p_p = jnp.concatenate(
        [P_vec.astype(jnp.float32), jnp.zeros((pad,), jnp.float32)])

    zeros = jnp.zeros((N_PAD, 128), jnp.float32)

    r1, y1 = _mm1(x, w1, bias1.reshape(1, D))
    y1p = lax.bitcast_convert_type(
        y1.reshape(N, B * D // 2, 2), jnp.int32)
    acc1, deg1 = _edge_l1(y1p, src_p, dst_p, et_p, p_p,
                          comp1.reshape(R * B), zeros)
    dcol = deg1[:, :, :32].reshape(NC, DEG_ROWS * 32)[:, :N]
    d0b = jnp.broadcast_to(dcol[0][:, None], (N, 16))
    d1b = jnp.broadcast_to(dcol[1][:, None], (N, 16))
    r2, y2 = _mm2(r1, acc1[0, :N], acc1[1, :N], d0b, d1b, w2,
                  bias2.reshape(1, C))
    (acc2,) = _edge_l2(y2, src_p, dst_p, et_p, p_p,
                       comp2.reshape(R * B), zeros)
    return _final(r2, acc2[0, :N, :C], acc2[1, :N, :C], d0b, d1b)


# speedup vs baseline: 1.3086x; 1.3086x over previous
import functools

import jax
import jax.numpy as jnp
from jax import lax
from jax.experimental import pallas as pl
from jax.experimental.pallas import tpu as pltpu
from jax.experimental.pallas import tpu_sc as plsc

N = 10000
E = 160000
D = 128
B = 8
R = 92
C = 16

NC = 2
NS = 16
NW = NC * NS
DUMP = N
N_PAD = 10112
RPS = N_PAD // NS
DEG_ROWS = 320
E_PAD = 163840
EPW = E_PAD // NW

ROW_BLK = 1000


def _make_edge_kernel(tw, K, CB, with_deg, table_bf16=False):
    G = K // 16
    dw = tw // B
    NB = EPW // K
    NCH = NB // CB
    CW = CB * K
    mesh = plsc.VectorSubcoreMesh(core_axis_name="c", subcore_axis_name="s")

    out_type = [jax.ShapeDtypeStruct((NC, N_PAD, 128), jnp.float32)]
    scratch = [
        pltpu.VMEM((2, CW), jnp.int32),
        pltpu.VMEM((2, CW), jnp.int32),
        pltpu.VMEM((2, CW), jnp.int32),
        pltpu.VMEM((2, CW), jnp.float32),
        pltpu.VMEM((R * B,), jnp.float32),
        pltpu.VMEM((2, K, tw // 2) if table_bf16 else (2, K, tw),
                   jnp.int32 if table_bf16 else jnp.float32),
        pltpu.VMEM((2, K, 128), jnp.float32),
        pltpu.VMEM((B, K), jnp.float32),
        pltpu.VMEM_SHARED((N_PAD, 128), jnp.float32),
        pltpu.SemaphoreType.DMA((2,)),
        pltpu.SemaphoreType.DMA((2,)),
    ]
    if with_deg:
        out_type.append(jax.ShapeDtypeStruct((NC, DEG_ROWS, 128), jnp.float32))
        scratch.append(pltpu.VMEM((2, K, 128), jnp.float32))
        scratch.append(pltpu.VMEM_SHARED((DEG_ROWS, 128), jnp.float32))

    @functools.partial(
        pl.kernel,
        out_type=out_type,
        mesh=mesh,
        compiler_params=pltpu.CompilerParams(needs_layout_passes=False),
        scratch_types=scratch,
    )
    def edge_kernel(y_hbm, src_hbm, dst_hbm, et_hbm, p_hbm, comp_hbm,
                    zeros_hbm, *refs):
        if with_deg:
            (out_hbm, deg_hbm, src_v, dst_v, et_v, p_v, comp_v, gbuf, mbuf,
             cbuf, acc_sh, gsem, ssem, dbuf, deg_sh) = refs
        else:
            (out_hbm, src_v, dst_v, et_v, p_v, comp_v, gbuf, mbuf,
             cbuf, acc_sh, gsem, ssem) = refs
        cid = lax.axis_index("c")
        sid = lax.axis_index("s")
        wid = cid * NS + sid
        ebase = wid * EPW

        def stage_chunk(ch, pslot):
            off = pl.multiple_of(ebase + ch * CW, 8)
            pltpu.sync_copy(src_hbm.at[pl.ds(off, CW)], src_v.at[pslot])
            pltpu.sync_copy(dst_hbm.at[pl.ds(off, CW)], dst_v.at[pslot])
            pltpu.sync_copy(et_hbm.at[pl.ds(off, CW)], et_v.at[pslot])
            pltpu.sync_copy(p_hbm.at[pl.ds(off, CW)], p_v.at[pslot])

        def eslice(j, g):
            ch = j // CB
            pslot = lax.rem(ch, 2)
            off = pl.multiple_of((j - ch * CB) * K + g * 16, 8)
            return pslot, off

        pltpu.sync_copy(comp_hbm, comp_v)
        pltpu.sync_copy(zeros_hbm.at[pl.ds(sid * RPS, RPS)],
                        acc_sh.at[pl.ds(sid * RPS, RPS)])
        if with_deg:
            @pl.when(sid == 0)
            def _():
                pltpu.sync_copy(zeros_hbm.at[pl.ds(0, DEG_ROWS)], deg_sh)

        zer = jnp.zeros((16,), jnp.float32)

        def zero_body(e, _):
            for s in range(2):
                for k in range(8):
                    mbuf[s, e, pl.ds(k * 16, 16)] = zer
                    if with_deg:
                        dbuf[s, e, pl.ds(k * 16, 16)] = zer
            return 0
        lax.fori_loop(0, K, zero_body, 0)

        def start_gather(j, slot):
            for g in range(G):
                pslot, off = eslice(j, g)
                ivec = src_v[pslot, pl.ds(off, 16)]
                pltpu.async_copy(y_hbm.at[ivec],
                                 gbuf.at[slot, pl.ds(g * 16, 16)],
                                 gsem.at[slot])

        def wait_gather(slot):
            for g in range(G):
                dummy = src_v[0, pl.ds(g * 16, 16)]
                pltpu.make_async_copy(y_hbm.at[dummy],
                                      gbuf.at[slot, pl.ds(g * 16, 16)],
                                      gsem.at[slot]).wait()

        def compute_batch(j, slot):
            cvecs = None
            for g in range(G):
                pslot, off = eslice(j, g)
                et_g = et_v[pslot, pl.ds(off, 16)].astype(jnp.float32)
                p_g = p_v[pslot, pl.ds(off, 16)]
                pf = et_g * p_g + 91.0 * (1.0 - p_g)
                pf = jnp.minimum(jnp.maximum(pf, 0.0), 91.0) + 0.5
                rel = pf.astype(jnp.int32) * B
                if K == 16:
                    cvecs = [plsc.load_gather(comp_v, [rel + b])
                             for b in range(B)]
                else:
                    for b in range(B):
                        cbuf[b, pl.ds(g * 16, 16)] = plsc.load_gather(
                            comp_v, [rel + b])

            @plsc.parallel_loop(0, K, step=1, unroll=2, carry=jnp.int32(0))
            def edge_body(e, _):
                if K == 16:
                    lane = e
                else:
                    cbase = pl.multiple_of(e & ~15, 8)
                    lane = e & 15
                lanes = jnp.full((16, 1), lane, jnp.int32)
                dnums = lax.GatherDimensionNumbers(
                    offset_dims=(), collapsed_slice_dims=(0,),
                    start_index_map=(0,))

                def bc(b):
                    src = (cvecs[b] if K == 16
                           else cbuf[b, pl.ds(cbase, 16)])
                    return lax.gather(
                        src, lanes, dnums, (1,),
                        mode=lax.GatherScatterMode.PROMISE_IN_BOUNDS)

                cbs = [bc(b) for b in range(B)]
                if table_bf16:
                    himask = jnp.full((16,), -65536, jnp.int32)
                    m = [None] * (dw // 16)
                    for b in range(B):
                        for t in range(dw // 32):
                            v = gbuf[slot, e, pl.ds(b * dw // 2 + t * 16, 16)]
                            av = plsc.bitcast(
                                lax.shift_left(v, jnp.full((16,), 16,
                                                           jnp.int32)),
                                jnp.float32)
                            bv = plsc.bitcast(v & himask, jnp.float32)
                            if b == 0:
                                m[2 * t] = cbs[0] * av
                                m[2 * t + 1] = cbs[0] * bv
                            else:
                                m[2 * t] = m[2 * t] + cbs[b] * av
                                m[2 * t + 1] = m[2 * t + 1] + cbs[b] * bv
                    for k in range(dw // 16):
                        mbuf[slot, e, pl.ds(k * 16, 16)] = m[k]
                else:
                    for k in range(dw // 16):
                        m = cbs[0] * gbuf[slot, e, pl.ds(k * 16, 16)]
                        for b in range(1, B):
                            m = m + cbs[b] * gbuf[slot, e,
                                                  pl.ds(b * dw + k * 16, 16)]
                        mbuf[slot, e, pl.ds(k * 16, 16)] = m
                if with_deg:
                    dbuf[slot, e, pl.ds(0, 16)] = zer
                    dbuf[slot, e, pl.ds(16, 16)] = zer
                return _

        ones_v = jnp.full((16,), 1.0, jnp.float32)
        lane_iota = lax.iota(jnp.int32, 16)

        def scatter_batch(j, slot):
            slotv = jnp.full((16,), slot, jnp.int32)
            for g in range(G):
                pslot, off = eslice(j, g)
                dvec = dst_v[pslot, pl.ds(off, 16)]
                if with_deg:
                    evec = lane_iota + (g * 16)
                    colv = dvec & 31
                    plsc.store_scatter(dbuf, [slotv, evec, colv], ones_v)
                pltpu.async_copy(mbuf.at[slot, pl.ds(g * 16, 16)],
                                 acc_sh.at[dvec], ssem.at[slot], add=True)
                if with_deg:
                    drow = lax.shift_right_logical(dvec, 5)
                    pltpu.async_copy(dbuf.at[slot, pl.ds(g * 16, 16)],
                                     deg_sh.at[drow], ssem.at[slot], add=True)

        def wait_scatter(slot):
            n_waits = G * (2 if with_deg else 1)
            for _w in range(n_waits):
                pltpu.make_async_copy(mbuf.at[slot, pl.ds(0, 16)],
                                      acc_sh.at[lane_iota],
                                      ssem.at[slot]).wait()

        plsc.subcore_barrier()

        stage_chunk(0, 0)
        start_gather(0, 0)

        def pair_body(i, _):
            j = i * 2

            @pl.when(lax.rem(j, CB) == 0)
            def _():
                ch = j // CB

                @pl.when(ch + 1 < NCH)
                def _():
                    stage_chunk(ch + 1, lax.rem(ch + 1, 2))

            wait_gather(0)
            start_gather(j + 1, 1)

            @pl.when(i > 0)
            def _():
                wait_scatter(0)

            compute_batch(j, 0)
            scatter_batch(j, 0)
            wait_gather(1)

            @pl.when(j + 2 < NB)
            def _():
                start_gather(j + 2, 0)

            @pl.when(i > 0)
            def _():
                wait_scatter(1)

            compute_batch(j + 1, 1)
            scatter_batch(j + 1, 1)
            return 0

        lax.fori_loop(0, NB // 2, pair_body, 0)

        wait_scatter(0)
        wait_scatter(1)
        plsc.subcore_barrier()
        pltpu.sync_copy(acc_sh.at[pl.ds(sid * RPS, RPS)],
                        out_hbm.at[cid, pl.ds(sid * RPS, RPS)])
        if with_deg:
            @pl.when(sid == 0)
            def _():
                pltpu.sync_copy(deg_sh, deg_hbm.at[cid])

    return edge_kernel


_edge_l1 = _make_edge_kernel(tw=B * D, K=16, CB=40, with_deg=True)
_edge_l2 = _make_edge_kernel(tw=B * C, K=64, CB=16, with_deg=False)


def _dot(a, b):
    return lax.dot_general(a, b, (((1,), (0,)), ((), ())),
                           precision=lax.Precision.HIGHEST,
                           preferred_element_type=jnp.float32)


def _mm1_body(x_ref, w_ref, b_ref, r_ref, y_ref):
    m = _dot(x_ref[...], w_ref[...])
    r_ref[...] = m[:, :D] + b_ref[...]
    y_ref[...] = m[:, D:]


def _inv_deg(deg):
    return jnp.where(deg > 0, 1.0 / jnp.maximum(deg, 1.0), 0.0)


def _mm2_body(r1_ref, a0_ref, a1_ref, d0_ref, d1_ref, w_ref, b_ref,
              r2_ref, y2_ref):
    inv = _inv_deg(d0_ref[...][:, 0] + d1_ref[...][:, 0])
    a = a0_ref[...] + a1_ref[...]
    h = jnp.maximum(r1_ref[...] + inv[:, None] * a, 0.0)
    m = _dot(h, w_ref[...])
    r2_ref[...] = m[:, :C] + b_ref[...]
    y2_ref[...] = m[:, C:]


def _final_body(r2_ref, c0_ref, c1_ref, d0_ref, d1_ref, o_ref):
    inv = _inv_deg(d0_ref[...][:, 0] + d1_ref[...][:, 0])
    logits = r2_ref[...] + inv[:, None] * (c0_ref[...] + c1_ref[...])
    mx = jnp.max(logits, axis=1, keepdims=True)
    s = logits - mx
    o_ref[...] = s - jnp.log(jnp.sum(jnp.exp(s), axis=1, keepdims=True))


def _row_spec(w):
    return pl.BlockSpec((ROW_BLK, w), lambda i: (i, 0))


def _full_spec(h, w):
    return pl.BlockSpec((h, w), lambda i: (0, 0))


_GRID = N // ROW_BLK

_mm1 = pl.pallas_call(
    _mm1_body,
    grid=(_GRID,),
    in_specs=[_row_spec(D), _full_spec(D, D + B * D), _full_spec(1, D)],
    out_specs=[_row_spec(D), _row_spec(B * D)],
    out_shape=[jax.ShapeDtypeStruct((N, D), jnp.float32),
               jax.ShapeDtypeStruct((N, B * D), jnp.float32)],
)

_mm2 = pl.pallas_call(
    _mm2_body,
    grid=(_GRID,),
    in_specs=[_row_spec(D), _row_spec(D), _row_spec(D),
              _row_spec(16), _row_spec(16),
              _full_spec(D, C + B * C), _full_spec(1, C)],
    out_specs=[_row_spec(C), _row_spec(B * C)],
    out_shape=[jax.ShapeDtypeStruct((N, C), jnp.float32),
               jax.ShapeDtypeStruct((N, B * C), jnp.float32)],
)

_final = pl.pallas_call(
    _final_body,
    grid=(_GRID,),
    in_specs=[_row_spec(C), _row_spec(C), _row_spec(C),
              _row_spec(16), _row_spec(16)],
    out_specs=_row_spec(C),
    out_shape=jax.ShapeDtypeStruct((N, C), jnp.float32),
)


def kernel(sub_edge_index, sub_edge_type, P_vec, entity_emb, bases1, comp1,
           root1, bias1, bases2, comp2, root2, bias2):
    x = entity_emb
    w1 = jnp.concatenate(
        [root1, jnp.transpose(bases1, (1, 0, 2)).reshape(D, B * D)], axis=1)
    w2 = jnp.concatenate(
        [root2, jnp.transpose(bases2, (1, 0, 2)).reshape(D, B * C)], axis=1)

    pad = E_PAD - E
    src_p = jnp.concatenate(
        [sub_edge_index[0].astype(jnp.int32), jnp.zeros((pad,), jnp.int32)])
    dst_p = jnp.concatenate(
        [sub_edge_index[1].astype(jnp.int32), jnp.full((pad,), DUMP, jnp.int32)])
    et_p = jnp.concatenate(
        [sub_edge_type.astype(jnp.int32), jnp.zeros((pad,), jnp.int32)])
    p_p = jnp.concatenate(
        [P_vec.astype(jnp.float32), jnp.zeros((pad,), jnp.float32)])

    zeros = jnp.zeros((N_PAD, 128), jnp.float32)

    r1, y1 = _mm1(x, w1, bias1.reshape(1, D))
    acc1, deg1 = _edge_l1(y1, src_p, dst_p, et_p, p_p,
                          comp1.reshape(R * B), zeros)
    dcol = deg1[:, :, :32].reshape(NC, DEG_ROWS * 32)[:, :N]
    d0b = jnp.broadcast_to(dcol[0][:, None], (N, 16))
    d1b = jnp.broadcast_to(dcol[1][:, None], (N, 16))
    r2, y2 = _mm2(r1, acc1[0, :N], acc1[1, :N], d0b, d1b, w2,
                  bias2.reshape(1, C))
    (acc2,) = _edge_l2(y2, src_p, dst_p, et_p, p_p,
                       comp2.reshape(R * B), zeros)
    return _final(r2, acc2[0, :N, :C], acc2[1, :N, :C], d0b, d1b)


# speedup vs baseline: 1.3092x; 1.0005x over previous
import functools

import jax
import jax.numpy as jnp
from jax import lax
from jax.experimental import pallas as pl
from jax.experimental.pallas import tpu as pltpu
from jax.experimental.pallas import tpu_sc as plsc

N = 10000
E = 160000
D = 128
B = 8
R = 92
C = 16

NC = 2
NS = 16
NW = NC * NS
DUMP = N
N_PAD = 10112
RPS = N_PAD // NS
DEG_ROWS = 320
E_PAD = 163840
EPW = E_PAD // NW

ROW_BLK = 1000


def _make_edge_kernel(tw, K, CB, with_deg):
    G = K // 16
    dw = tw // B
    NB = EPW // K
    NCH = NB // CB
    CW = CB * K
    mesh = plsc.VectorSubcoreMesh(core_axis_name="c", subcore_axis_name="s")

    out_type = [jax.ShapeDtypeStruct((NC, N_PAD, 128), jnp.float32)]
    scratch = [
        pltpu.VMEM((2, CW), jnp.int32),
        pltpu.VMEM((2, CW), jnp.int32),
        pltpu.VMEM((2, CW), jnp.int32),
        pltpu.VMEM((2, CW), jnp.float32),
        pltpu.VMEM((R * B,), jnp.float32),
        pltpu.VMEM((2, K, tw), jnp.float32),
        pltpu.VMEM((2, K, 128), jnp.float32),
        pltpu.VMEM((B, K), jnp.float32),
        pltpu.VMEM_SHARED((N_PAD, 128), jnp.float32),
        pltpu.SemaphoreType.DMA((2,)),
        pltpu.SemaphoreType.DMA((2,)),
    ]
    if with_deg:
        out_type.append(jax.ShapeDtypeStruct((NC, DEG_ROWS, 128), jnp.float32))
        scratch.append(pltpu.VMEM((2, K, 128), jnp.float32))
        scratch.append(pltpu.VMEM_SHARED((DEG_ROWS, 128), jnp.float32))

    @functools.partial(
        pl.kernel,
        out_type=out_type,
        mesh=mesh,
        compiler_params=pltpu.CompilerParams(needs_layout_passes=False),
        scratch_types=scratch,
    )
    def edge_kernel(y_hbm, src_hbm, dst_hbm, et_hbm, p_hbm, comp_hbm,
                    zeros_hbm, *refs):
        if with_deg:
            (out_hbm, deg_hbm, src_v, dst_v, et_v, p_v, comp_v, gbuf, mbuf,
             cbuf, acc_sh, gsem, ssem, dbuf, deg_sh) = refs
        else:
            (out_hbm, src_v, dst_v, et_v, p_v, comp_v, gbuf, mbuf,
             cbuf, acc_sh, gsem, ssem) = refs
        cid = lax.axis_index("c")
        sid = lax.axis_index("s")
        wid = cid * NS + sid
        ebase = wid * EPW

        def stage_chunk(ch, pslot):
            off = pl.multiple_of(ebase + ch * CW, 8)
            pltpu.sync_copy(src_hbm.at[pl.ds(off, CW)], src_v.at[pslot])
            pltpu.sync_copy(dst_hbm.at[pl.ds(off, CW)], dst_v.at[pslot])
            pltpu.sync_copy(et_hbm.at[pl.ds(off, CW)], et_v.at[pslot])
            pltpu.sync_copy(p_hbm.at[pl.ds(off, CW)], p_v.at[pslot])

        def eslice(j, g):
            ch = j // CB
            pslot = lax.rem(ch, 2)
            off = pl.multiple_of((j - ch * CB) * K + g * 16, 8)
            return pslot, off

        pltpu.sync_copy(comp_hbm, comp_v)
        pltpu.sync_copy(zeros_hbm.at[pl.ds(sid * RPS, RPS)],
                        acc_sh.at[pl.ds(sid * RPS, RPS)])
        if with_deg:
            @pl.when(sid == 0)
            def _():
                pltpu.sync_copy(zeros_hbm.at[pl.ds(0, DEG_ROWS)], deg_sh)

        zer = jnp.zeros((16,), jnp.float32)

        def zero_body(e, _):
            for s in range(2):
                for k in range(8):
                    mbuf[s, e, pl.ds(k * 16, 16)] = zer
                    if with_deg:
                        dbuf[s, e, pl.ds(k * 16, 16)] = zer
            return 0
        lax.fori_loop(0, K, zero_body, 0)

        def start_gather(j, slot):
            for g in range(G):
                pslot, off = eslice(j, g)
                ivec = src_v[pslot, pl.ds(off, 16)]
                pltpu.async_copy(y_hbm.at[ivec],
                                 gbuf.at[slot, pl.ds(g * 16, 16)],
                                 gsem.at[slot])

        def wait_gather(slot):
            for g in range(G):
                dummy = src_v[0, pl.ds(g * 16, 16)]
                pltpu.make_async_copy(y_hbm.at[dummy],
                                      gbuf.at[slot, pl.ds(g * 16, 16)],
                                      gsem.at[slot]).wait()

        def compute_batch(j, slot):
            cvecs = None
            for g in range(G):
                pslot, off = eslice(j, g)
                et_g = et_v[pslot, pl.ds(off, 16)].astype(jnp.float32)
                p_g = p_v[pslot, pl.ds(off, 16)]
                pf = et_g * p_g + 91.0 * (1.0 - p_g)
                pf = jnp.minimum(jnp.maximum(pf, 0.0), 91.0) + 0.5
                rel = pf.astype(jnp.int32) * B
                if K == 16:
                    cvecs = [plsc.load_gather(comp_v, [rel + b])
                             for b in range(B)]
                else:
                    for b in range(B):
                        cbuf[b, pl.ds(g * 16, 16)] = plsc.load_gather(
                            comp_v, [rel + b])

            @plsc.parallel_loop(0, K, step=1, unroll=2, carry=jnp.int32(0))
            def edge_body(e, _):
                if K == 16:
                    lane = e
                else:
                    cbase = pl.multiple_of(e & ~15, 8)
                    lane = e & 15
                lanes = jnp.full((16, 1), lane, jnp.int32)
                dnums = lax.GatherDimensionNumbers(
                    offset_dims=(), collapsed_slice_dims=(0,),
                    start_index_map=(0,))

                def bc(b):
                    src = (cvecs[b] if K == 16
                           else cbuf[b, pl.ds(cbase, 16)])
                    return lax.gather(
                        src, lanes, dnums, (1,),
                        mode=lax.GatherScatterMode.PROMISE_IN_BOUNDS)

                cbs = [bc(b) for b in range(B)]
                for k in range(dw // 16):
                    m = cbs[0] * gbuf[slot, e, pl.ds(k * 16, 16)]
                    for b in range(1, B):
                        m = m + cbs[b] * gbuf[slot, e,
                                              pl.ds(b * dw + k * 16, 16)]
                    mbuf[slot, e, pl.ds(k * 16, 16)] = m
                if with_deg:
                    dbuf[slot, e, pl.ds(0, 16)] = zer
                    dbuf[slot, e, pl.ds(16, 16)] = zer
                return _

        ones_v = jnp.full((16,), 1.0, jnp.float32)
        lane_iota = lax.iota(jnp.int32, 16)

        def scatter_batch(j, slot):
            slotv = jnp.full((16,), slot, jnp.int32)
            for g in range(G):
                pslot, off = eslice(j, g)
                dvec = dst_v[pslot, pl.ds(off, 16)]
                if with_deg:
                    evec = lane_iota + (g * 16)
                    colv = dvec & 31
                    plsc.store_scatter(dbuf, [slotv, evec, colv], ones_v)
                pltpu.async_copy(mbuf.at[slot, pl.ds(g * 16, 16)],
                                 acc_sh.at[dvec], ssem.at[slot], add=True)
                if with_deg:
                    drow = lax.shift_right_logical(dvec, 5)
                    pltpu.async_copy(dbuf.at[slot, pl.ds(g * 16, 16)],
                                     deg_sh.at[drow], ssem.at[slot], add=True)

        def wait_scatter(slot):
            n_waits = G * (2 if with_deg else 1)
            for _w in range(n_waits):
                pltpu.make_async_copy(mbuf.at[slot, pl.ds(0, 16)],
                                      acc_sh.at[lane_iota],
                                      ssem.at[slot]).wait()

        plsc.subcore_barrier()

        stage_chunk(0, 0)
        start_gather(0, 0)

        def pair_body(i, _):
            j = i * 2

            @pl.when(lax.rem(j, CB) == 0)
            def _():
                ch = j // CB

                @pl.when(ch + 1 < NCH)
                def _():
                    stage_chunk(ch + 1, lax.rem(ch + 1, 2))

            wait_gather(0)
            start_gather(j + 1, 1)

            @pl.when(i > 0)
            def _():
                wait_scatter(0)

            compute_batch(j, 0)
            scatter_batch(j, 0)
            wait_gather(1)

            @pl.when(j + 2 < NB)
            def _():
                start_gather(j + 2, 0)

            @pl.when(i > 0)
            def _():
                wait_scatter(1)

            compute_batch(j + 1, 1)
            scatter_batch(j + 1, 1)
            return 0

        lax.fori_loop(0, NB // 2, pair_body, 0)

        wait_scatter(0)
        wait_scatter(1)
        plsc.subcore_barrier()
        pltpu.sync_copy(acc_sh.at[pl.ds(sid * RPS, RPS)],
                        out_hbm.at[cid, pl.ds(sid * RPS, RPS)])
        if with_deg:
            @pl.when(sid == 0)
            def _():
                pltpu.sync_copy(deg_sh, deg_hbm.at[cid])

    return edge_kernel


_edge_l1 = _make_edge_kernel(tw=B * D, K=16, CB=40, with_deg=True)
_edge_l2 = _make_edge_kernel(tw=B * C, K=64, CB=16, with_deg=False)


def _dot(a, b):
    return lax.dot_general(a, b, (((1,), (0,)), ((), ())),
                           precision=lax.Precision.HIGHEST,
                           preferred_element_type=jnp.float32)


def _mm1_body(x_ref, w_ref, b_ref, r_ref, y_ref):
    m = _dot(x_ref[...], w_ref[...])
    r_ref[...] = m[:, :D] + b_ref[...]
    y_ref[...] = m[:, D:]


def _inv_deg(deg):
    return jnp.where(deg > 0, 1.0 / jnp.maximum(deg, 1.0), 0.0)


def _mm2_body(r1_ref, a0_ref, a1_ref, d0_ref, d1_ref, w_ref, b_ref,
              r2_ref, y2_ref):
    inv = _inv_deg(d0_ref[...][:, 0] + d1_ref[...][:, 0])
    a = a0_ref[...] + a1_ref[...]
    h = jnp.maximum(r1_ref[...] + inv[:, None] * a, 0.0)
    m = _dot(h, w_ref[...])
    r2_ref[...] = m[:, :C] + b_ref[...]
    y2_ref[...] = m[:, C:]


def _final_body(r2_ref, c0_ref, c1_ref, d0_ref, d1_ref, o_ref):
    inv = _inv_deg(d0_ref[...][:, 0] + d1_ref[...][:, 0])
    logits = r2_ref[...] + inv[:, None] * (c0_ref[...] + c1_ref[...])
    mx = jnp.max(logits, axis=1, keepdims=True)
    s = logits - mx
    o_ref[...] = s - jnp.log(jnp.sum(jnp.exp(s), axis=1, keepdims=True))


def _row_spec(w):
    return pl.BlockSpec((ROW_BLK, w), lambda i: (i, 0))


def _full_spec(h, w):
    return pl.BlockSpec((h, w), lambda i: (0, 0))


_GRID = N // ROW_BLK

_mm1 = pl.pallas_call(
    _mm1_body,
    grid=(_GRID,),
    in_specs=[_row_spec(D), _full_spec(D, D + B * D), _full_spec(1, D)],
    out_specs=[_row_spec(D), _row_spec(B * D)],
    out_shape=[jax.ShapeDtypeStruct((N, D), jnp.float32),
               jax.ShapeDtypeStruct((N, B * D), jnp.float32)],
)

_mm2 = pl.pallas_call(
    _mm2_body,
    grid=(_GRID,),
    in_specs=[_row_spec(D), _row_spec(D), _row_spec(D),
              _row_spec(16), _row_spec(16),
              _full_spec(D, C + B * C), _full_spec(1, C)],
    out_specs=[_row_spec(C), _row_spec(B * C)],
    out_shape=[jax.ShapeDtypeStruct((N, C), jnp.float32),
               jax.ShapeDtypeStruct((N, B * C), jnp.float32)],
)

_final = pl.pallas_call(
    _final_body,
    grid=(_GRID,),
    in_specs=[_row_spec(C), _row_spec(C), _row_spec(C),
              _row_spec(16), _row_spec(16)],
    out_specs=_row_spec(C),
    out_shape=jax.ShapeDtypeStruct((N, C), jnp.float32),
)


def kernel(sub_edge_index, sub_edge_type, P_vec, entity_emb, bases1, comp1,
           root1, bias1, bases2, comp2, root2, bias2):
    x = entity_emb
    w1 = jnp.concatenate(
        [root1, jnp.transpose(bases1, (1, 0, 2)).reshape(D, B * D)], axis=1)
    w2 = jnp.concatenate(
        [root2, jnp.transpose(bases2, (1, 0, 2)).reshape(D, B * C)], axis=1)

    pad = E_PAD - E
    src_p = jnp.concatenate(
        [sub_edge_index[0].astype(jnp.int32), jnp.zeros((pad,), jnp.int32)])
    dst_p = jnp.concatenate(
        [sub_edge_index[1].astype(jnp.int32), jnp.full((pad,), DUMP, jnp.int32)])
    et_p = jnp.concatenate(
        [sub_edge_type.astype(jnp.int32), jnp.zeros((pad,), jnp.int32)])
    p_p = jnp.concatenate(
        [P_vec.astype(jnp.float32), jnp.zeros((pad,), jnp.float32)])

    zeros = jnp.zeros((N_PAD, 128), jnp.float32)

    r1, y1 = _mm1(x, w1, bias1.reshape(1, D))
    acc1, deg1 = _edge_l1(y1, src_p, dst_p, et_p, p_p,
                          comp1.reshape(R * B), zeros)
    dcol = deg1[:, :, :32].reshape(NC, DEG_ROWS * 32)[:, :N]
    d0b = jnp.broadcast_to(dcol[0][:, None], (N, 16))
    d1b = jnp.broadcast_to(dcol[1][:, None], (N, 16))
    r2, y2 = _mm2(r1, acc1[0, :N], acc1[1, :N], d0b, d1b, w2,
                  bias2.reshape(1, C))
    (acc2,) = _edge_l2(y2, src_p, dst_p, et_p, p_p,
                       comp2.reshape(R * B), zeros)
    return _final(r2, acc2[0, :N, :C], acc2[1, :N, :C], d0b, d1b)
